# baseline (speedup 1.0000x reference)
import jax
import jax.numpy as jnp
from jax import lax
from jax.experimental import pallas as pl
from jax.experimental.pallas import tpu as pltpu
from jax.experimental.pallas import tpu_sc as plsc

B = 16384
D_NUM = 13
N_CAT = 26
D_TOKEN = 32
N_TOK = D_NUM + N_CAT
TOTAL_CAT = 2600000
WIDE = 4 * D_TOKEN

NC = 2
NS = 16
NW = NC * NS
ROWS_W = B // NW
CB = 8
NCHUNK = ROWS_W // CB
NPAIR = NCHUNK // 2
IDX_CHUNK = CB * N_CAT
NVEC_IDX = IDX_CHUNK // 16
IDX_W = ROWS_W * N_CAT
XNUM_W = ROWS_W * 16


def _body(table, idxf, xnum, weight, bias, out,
          idxf_v, xnum_v, idx0, idx1, rows0, rows1, out0, out1,
          w_v, b_v, gsem0, gsem1, osem0, osem1):
  cid = lax.axis_index("c")
  sid = lax.axis_index("s")
  wid = sid * NC + cid

  bufs = ((idx0, rows0, out0, gsem0, osem0),
          (idx1, rows1, out1, gsem1, osem1))

  pltpu.sync_copy(idxf.at[pl.ds(wid * IDX_W, IDX_W)], idxf_v)
  pltpu.sync_copy(xnum.at[pl.ds(wid * XNUM_W, XNUM_W)], xnum_v)
  pltpu.sync_copy(weight, w_v)
  pltpu.sync_copy(bias, b_v)

  def fire(i, buf):
    idx_v, rows_v, _, gsem, _ = buf
    for v in range(NVEC_IDX):
      idx_v[pl.ds(v * 16, 16)] = (
          idxf_v[pl.ds(i * IDX_CHUNK + v * 16, 16)] & ((1 << 20) - 1))
    pltpu.async_copy(table.at[idx_v], rows_v, gsem)

  def process(i, buf, not_first_use):
    _, rows_v, out_v, gsem, osem = buf
    base = pl.multiple_of(wid * ROWS_W + i * CB, CB)
    pltpu.make_async_copy(table.at[pl.ds(0, IDX_CHUNK)], rows_v, gsem).wait()

    @pl.when(not_first_use)
    def _():
      pltpu.make_async_copy(out_v, out.at[pl.ds(0, CB)], osem).wait()

    def vb_body(vb, carry):
      p0 = i * IDX_CHUNK + vb * 16
      ivec = idxf_v[pl.ds(p0, 16)]
      for u in range(16):
        iu = ivec[u]
        col = lax.shift_right_logical(iu, 20) * D_TOKEN
        p = vb * 16 + u
        r = p // N_CAT
        c = p % N_CAT
        for h in range(2):
          out_v[r, D_NUM + c, pl.ds(h * 16, 16)] = (
              rows_v[p, pl.ds(col + h * 16, 16)]
              + b_v[D_NUM + c, pl.ds(h * 16, 16)])
      return carry

    lax.fori_loop(0, NVEC_IDX, vb_body, 0)

    def row_body(r, carry):
      row_vec = xnum_v[pl.ds((i * CB + r) * 16, 16)]
      for j in range(D_NUM):
        xs = jnp.broadcast_to(row_vec[j], (16,))
        for h in range(2):
          out_v[r, j, pl.ds(h * 16, 16)] = (
              w_v[j, pl.ds(h * 16, 16)] * xs + b_v[j, pl.ds(h * 16, 16)])
      return carry

    lax.fori_loop(0, CB, row_body, 0)
    pltpu.async_copy(out_v, out.at[pl.ds(base, CB)], osem)

  fire(0, bufs[0])

  def pair_body(i2, carry):
    i = i2 * 2
    fire(i + 1, bufs[1])
    process(i, bufs[0], i2 >= 1)

    @pl.when(i2 < NPAIR - 1)
    def _():
      fire(i + 2, bufs[0])
    process(i + 1, bufs[1], i2 >= 1)
    return carry

  lax.fori_loop(0, NPAIR, pair_body, 0)

  pltpu.make_async_copy(out0, out.at[pl.ds(0, CB)], osem0).wait()
  pltpu.make_async_copy(out1, out.at[pl.ds(0, CB)], osem1).wait()


@jax.jit
def _tokenizer(table_wide, idx_flat, xnum_flat, weight, bias):
  mesh = plsc.VectorSubcoreMesh(core_axis_name="c", subcore_axis_name="s")
  out = pl.kernel(
      _body,
      out_type=jax.ShapeDtypeStruct((B, N_TOK, D_TOKEN), jnp.float32),
      mesh=mesh,
      scratch_types=[
          pltpu.VMEM((IDX_W,), jnp.int32),
          pltpu.VMEM((XNUM_W,), jnp.float32),
          pltpu.VMEM((IDX_CHUNK,), jnp.int32),
          pltpu.VMEM((IDX_CHUNK,), jnp.int32),
          pltpu.VMEM((IDX_CHUNK, WIDE), jnp.float32),
          pltpu.VMEM((IDX_CHUNK, WIDE), jnp.float32),
          pltpu.VMEM((CB, N_TOK, D_TOKEN), jnp.float32),
          pltpu.VMEM((CB, N_TOK, D_TOKEN), jnp.float32),
          pltpu.VMEM((D_NUM, D_TOKEN), jnp.float32),
          pltpu.VMEM((N_TOK, D_TOKEN), jnp.float32),
          pltpu.SemaphoreType.DMA,
          pltpu.SemaphoreType.DMA,
          pltpu.SemaphoreType.DMA,
          pltpu.SemaphoreType.DMA,
      ],
      compiler_params=pltpu.CompilerParams(use_tc_tiling_on_sc=False),
  )(table_wide, idx_flat, xnum_flat, weight, bias)
  return out


QROWS = TOTAL_CAT // 4
_RQ = 2600
_RNB = QROWS // _RQ


def _repack_body(i0, i1, i2, i3, o_ref):
  o_ref[...] = jnp.concatenate(
      [i0[...], i1[...], i2[...], i3[...]], axis=1)


@jax.jit
def _repack(table):
  return pl.pallas_call(
      _repack_body,
      grid=(_RNB,),
      in_specs=[
          pl.BlockSpec((_RQ, D_TOKEN), lambda g, k=k: (g + k * _RNB, 0))
          for k in range(4)
      ],
      out_specs=pl.BlockSpec((_RQ, WIDE), lambda g: (g, 0)),
      out_shape=jax.ShapeDtypeStruct((QROWS, WIDE), jnp.float32),
  )(table, table, table, table)


def kernel(x_num, x_cat, weight, bias, category_embeddings, category_offsets):
  table_wide = _repack(category_embeddings)
  idx = (x_cat + category_offsets[None, :]).reshape(-1)
  idx_flat = (idx % QROWS) | ((idx // QROWS) << 20)
  xnum_flat = jnp.pad(x_num, ((0, 0), (0, 16 - D_NUM))).reshape(-1)
  return _tokenizer(table_wide, idx_flat, xnum_flat, weight, bias)


# speedup vs baseline: 1.0377x; 1.0377x over previous
import jax
import jax.numpy as jnp
from jax import lax
from jax.experimental import pallas as pl
from jax.experimental.pallas import tpu as pltpu
from jax.experimental.pallas import tpu_sc as plsc

B = 16384
D_NUM = 13
N_CAT = 26
D_TOKEN = 32
N_TOK = D_NUM + N_CAT
TOTAL_CAT = 2600000

NC = 2
NS = 16
NW = NC * NS
ROWS_W = B // NW
CB = 16
NCHUNK = ROWS_W // CB
NPAIR = NCHUNK // 2
IDX_CHUNK = CB * N_CAT
NVEC_IDX = IDX_CHUNK // 16
OFFS_TILE = 208
OUT_CHUNK = CB * N_TOK * D_TOKEN


def _body(table, xcat, xnum, weight, bias, offs, out,
          xcat0, xcat1, idx0, idx1, rows0, rows1, xnum0, xnum1,
          out0, out1, w_v, b_v, offs_v, gsem0, gsem1, osem0, osem1):
  cid = lax.axis_index("c")
  sid = lax.axis_index("s")
  wid = sid * NC + cid

  bufs = ((xcat0, idx0, rows0, xnum0, out0, gsem0, osem0),
          (xcat1, idx1, rows1, xnum1, out1, gsem1, osem1))

  pltpu.sync_copy(weight, w_v)
  pltpu.sync_copy(bias, b_v)
  pltpu.sync_copy(offs, offs_v)

  def stage_and_fire(i, buf):
    xcat_v, idx_v, rows_v, xnum_v, _, gsem, _ = buf
    base = pl.multiple_of(wid * ROWS_W + i * CB, CB)
    pltpu.sync_copy(xcat.at[pl.ds(base * N_CAT, IDX_CHUNK)], xcat_v)
    pltpu.sync_copy(xnum.at[pl.ds(base * 16, CB * 16)], xnum_v)
    for v in range(NVEC_IDX):
      vec = (xcat_v[pl.ds(v * 16, 16)]
             + offs_v[pl.ds((v % (OFFS_TILE // 16)) * 16, 16)])
      idx_v[pl.ds(v * 16, 16)] = vec
    pltpu.async_copy(table.at[idx_v], rows_v, gsem)

  def process(i, buf, not_first_use):
    _, _, rows_v, xnum_v, out_v, gsem, osem = buf
    base = pl.multiple_of(wid * ROWS_W + i * CB, CB)
    pltpu.make_async_copy(
        table.at[pl.ds(0, IDX_CHUNK)], rows_v, gsem).wait()

    @pl.when(not_first_use)
    def _():
      pltpu.make_async_copy(
          out_v, out.at[pl.ds(0, OUT_CHUNK)], osem).wait()

    def row_body(r, carry):
      for c in range(N_CAT):
        for h in range(2):
          out_v[pl.ds((r * N_TOK + D_NUM + c) * D_TOKEN + h * 16, 16)] = (
              rows_v[r * N_CAT + c, pl.ds(h * 16, 16)]
              + b_v[D_NUM + c, pl.ds(h * 16, 16)])
      row_vec = xnum_v[pl.ds(r * 16, 16)]
      for j in range(D_NUM):
        xs = jnp.broadcast_to(row_vec[j], (16,))
        for h in range(2):
          out_v[pl.ds((r * N_TOK + j) * D_TOKEN + h * 16, 16)] = (
              w_v[j, pl.ds(h * 16, 16)] * xs + b_v[j, pl.ds(h * 16, 16)])
      return carry

    lax.fori_loop(0, CB, row_body, 0)
    pltpu.async_copy(out_v, out.at[pl.ds(base * N_TOK * D_TOKEN, OUT_CHUNK)],
                     osem)

  stage_and_fire(0, bufs[0])

  def pair_body(i2, carry):
    i = i2 * 2
    stage_and_fire(i + 1, bufs[1])
    process(i, bufs[0], i2 >= 1)

    @pl.when(i2 < NPAIR - 1)
    def _():
      stage_and_fire(i + 2, bufs[0])
    process(i + 1, bufs[1], i2 >= 1)
    return carry

  lax.fori_loop(0, NPAIR, pair_body, 0)

  pltpu.make_async_copy(out0, out.at[pl.ds(0, OUT_CHUNK)], osem0).wait()
  pltpu.make_async_copy(out1, out.at[pl.ds(0, OUT_CHUNK)], osem1).wait()


@jax.jit
def _tokenizer(table, xcat_flat, xnum_flat, weight, bias, offs_tiled):
  mesh = plsc.VectorSubcoreMesh(core_axis_name="c", subcore_axis_name="s")
  out = pl.kernel(
      _body,
      out_type=jax.ShapeDtypeStruct((B * N_TOK * D_TOKEN,), jnp.float32),
      mesh=mesh,
      scratch_types=[
          pltpu.VMEM((IDX_CHUNK,), jnp.int32),
          pltpu.VMEM((IDX_CHUNK,), jnp.int32),
          pltpu.VMEM((IDX_CHUNK,), jnp.int32),
          pltpu.VMEM((IDX_CHUNK,), jnp.int32),
          pltpu.VMEM((IDX_CHUNK, D_TOKEN), jnp.float32),
          pltpu.VMEM((IDX_CHUNK, D_TOKEN), jnp.float32),
          pltpu.VMEM((CB * 16,), jnp.float32),
          pltpu.VMEM((CB * 16,), jnp.float32),
          pltpu.VMEM((OUT_CHUNK,), jnp.float32),
          pltpu.VMEM((OUT_CHUNK,), jnp.float32),
          pltpu.VMEM((D_NUM, D_TOKEN), jnp.float32),
          pltpu.VMEM((N_TOK, D_TOKEN), jnp.float32),
          pltpu.VMEM((OFFS_TILE,), jnp.int32),
          pltpu.SemaphoreType.DMA,
          pltpu.SemaphoreType.DMA,
          pltpu.SemaphoreType.DMA,
          pltpu.SemaphoreType.DMA,
      ],
      compiler_params=pltpu.CompilerParams(use_tc_tiling_on_sc=False),
  )(table, xcat_flat, xnum_flat, weight, bias, offs_tiled)
  return out.reshape(B, N_TOK, D_TOKEN)


def kernel(x_num, x_cat, weight, bias, category_embeddings, category_offsets):
  xcat_flat = x_cat.reshape(-1)
  xnum_flat = jnp.pad(x_num, ((0, 0), (0, 16 - D_NUM))).reshape(-1)
  offs_tiled = jnp.tile(category_offsets, OFFS_TILE // N_CAT)
  return _tokenizer(category_embeddings, xcat_flat, xnum_flat,
                    weight, bias, offs_tiled)
